# Initial kernel scaffold; baseline (speedup 1.0000x reference)
import jax
import jax.numpy as jnp
from jax.experimental import pallas as pl


def kernel(x, hyperedge_index, W, b):
    raise NotImplementedError("write your pallas kernel here")


# speedup vs baseline: 5.4719x; 5.4719x over previous
import functools

import jax
import jax.numpy as jnp
from jax import lax
from jax.experimental import pallas as pl
from jax.experimental.pallas import tpu as pltpu
from jax.experimental.pallas import tpu_sc as plsc

N_NODES = 10000
N_EDGES = 10000
N_INC = 320000
CH = 128

NC = 2
NS = 16
NW = NC * NS
LANES = 16

INC_PER_W = N_INC // NW
CHUNK = 80
N_CHUNKS = INC_PER_W // CHUNK
ACC_N = 10240
ROWS_PER_TILE = ACC_N // NS
ZB_ROWS = 128

_mesh = plsc.VectorSubcoreMesh(core_axis_name="c", subcore_axis_name="s")
_sc_params = pltpu.CompilerParams(needs_layout_passes=False)


@functools.partial(
    pl.kernel,
    mesh=_mesh,
    out_type=[
        jax.ShapeDtypeStruct((NW, 1, N_NODES), jnp.float32),
        jax.ShapeDtypeStruct((NW, 1, N_EDGES), jnp.float32),
    ],
    scratch_types=[
        pltpu.VMEM((INC_PER_W,), jnp.int32),
        pltpu.VMEM((N_NODES,), jnp.float32),
    ],
    compiler_params=_sc_params,
)
def _hist(nidx_hbm, hidx_hbm, dn_out, de_out, idx_v, cnt_v):
    wid = lax.axis_index("s") * NC + lax.axis_index("c")
    ones = jnp.ones((LANES,), jnp.float32)
    zeros = jnp.zeros((LANES,), jnp.float32)

    for src, out in ((nidx_hbm, dn_out), (hidx_hbm, de_out)):
        @pl.loop(0, N_NODES, step=LANES)
        def _zero(i):
            cnt_v[pl.ds(i, LANES)] = zeros

        pltpu.sync_copy(src.at[wid, 0], idx_v)

        @pl.loop(0, INC_PER_W, step=LANES)
        def _accum(i):
            idx = idx_v[pl.ds(i, LANES)]
            plsc.addupdate_scatter(cnt_v, [idx], ones)

        pltpu.sync_copy(cnt_v, out.at[wid, 0])


@functools.partial(
    pl.kernel,
    mesh=_mesh,
    out_type=jax.ShapeDtypeStruct((NC, ACC_N, CH), jnp.float32),
    scratch_types=[
        pltpu.VMEM((1, CHUNK), jnp.int32),
        pltpu.VMEM((1, CHUNK), jnp.int32),
        pltpu.VMEM((CHUNK, CH), jnp.float32),
        pltpu.VMEM((ZB_ROWS, CH), jnp.float32),
        pltpu.VMEM_SHARED((ACC_N, CH), jnp.float32),
        pltpu.SemaphoreType.DMA,
    ],
    compiler_params=_sc_params,
)
def _segpass(table_hbm, gidx_hbm, sidx_hbm, out_hbm,
             gi_v, si_v, rows_v, zb_v, acc_sh, sem):
    c = lax.axis_index("c")
    s = lax.axis_index("s")
    wid = s * NC + c
    zeros = jnp.zeros((LANES,), jnp.float32)

    @pl.loop(0, ZB_ROWS)
    def _zrow(i):
        @pl.loop(0, CH, step=LANES)
        def _zcol(j):
            zb_v[i, pl.ds(j, LANES)] = zeros

    @pl.loop(0, ROWS_PER_TILE // ZB_ROWS)
    def _zcp(k):
        pltpu.sync_copy(zb_v, acc_sh.at[pl.ds(s * ROWS_PER_TILE + k * ZB_ROWS,
                                              ZB_ROWS)])

    plsc.subcore_barrier()

    @pl.loop(0, N_CHUNKS)
    def _chunk(ci):
        blk = wid * N_CHUNKS + ci
        pltpu.sync_copy(gidx_hbm.at[blk, 0], gi_v.at[0])
        pltpu.sync_copy(sidx_hbm.at[blk, 0], si_v.at[0])
        pltpu.async_copy(table_hbm.at[gi_v.at[0]], rows_v, sem).wait()
        pltpu.sync_copy(rows_v, acc_sh.at[si_v.at[0]], add=True)

    plsc.subcore_barrier()

    pltpu.sync_copy(acc_sh.at[pl.ds(s * ROWS_PER_TILE, ROWS_PER_TILE)],
                    out_hbm.at[c, pl.ds(s * ROWS_PER_TILE, ROWS_PER_TILE)])


_BM = 1000


def _scales_body(dnp_ref, dep_ref, dns_ref, dei_ref):
    dn = jnp.sum(dnp_ref[...].T, axis=1, keepdims=True)
    dns_ref[...] = jnp.where(dn > 0, lax.rsqrt(jnp.maximum(dn, 1e-12)), 0.0)
    de = jnp.sum(dep_ref[...].T, axis=1, keepdims=True)
    dei_ref[...] = jnp.where(de > 0, 1.0 / jnp.maximum(de, 1e-12), 0.0)


def _scales(dn_p, de_p):
    return pl.pallas_call(
        _scales_body,
        out_shape=[jax.ShapeDtypeStruct((N_NODES, 1), jnp.float32),
                   jax.ShapeDtypeStruct((N_EDGES, 1), jnp.float32)],
    )(dn_p, de_p)


def _proj_body(x_ref, wt_ref, b_ref, dns_ref, h_ref):
    xw = jnp.dot(x_ref[...], wt_ref[...],
                 preferred_element_type=jnp.float32) + b_ref[...]
    h_ref[...] = xw * dns_ref[...]


def _proj(x, wt, b2, dn_s):
    return pl.pallas_call(
        _proj_body,
        grid=(N_NODES // _BM,),
        in_specs=[
            pl.BlockSpec((_BM, CH), lambda i: (i, 0)),
            pl.BlockSpec((CH, CH), lambda i: (0, 0)),
            pl.BlockSpec((1, CH), lambda i: (0, 0)),
            pl.BlockSpec((_BM, 1), lambda i: (i, 0)),
        ],
        out_specs=pl.BlockSpec((_BM, CH), lambda i: (i, 0)),
        out_shape=jax.ShapeDtypeStruct((N_NODES, CH), jnp.float32),
    )(x, wt, b2, dn_s)


def _combine_body(relu, p_ref, s_ref, o_ref):
    tot = (p_ref[0] + p_ref[1]) * s_ref[...]
    o_ref[...] = jnp.maximum(tot, 0.0) if relu else tot


def _combine(p, s, relu):
    return pl.pallas_call(
        functools.partial(_combine_body, relu),
        grid=(N_NODES // _BM,),
        in_specs=[
            pl.BlockSpec((NC, _BM, CH), lambda i: (0, i, 0)),
            pl.BlockSpec((_BM, 1), lambda i: (i, 0)),
        ],
        out_specs=pl.BlockSpec((_BM, CH), lambda i: (i, 0)),
        out_shape=jax.ShapeDtypeStruct((N_NODES, CH), jnp.float32),
    )(p, s)


def kernel(x, hyperedge_index, W, b):
    nidx = hyperedge_index[0]
    hidx = hyperedge_index[1]
    nidx_w = nidx.reshape(NW, 1, INC_PER_W)
    hidx_w = hidx.reshape(NW, 1, INC_PER_W)
    nidx_c = nidx.reshape(NW * N_CHUNKS, 1, CHUNK)
    hidx_c = hidx.reshape(NW * N_CHUNKS, 1, CHUNK)
    wt = W.T
    b2 = b.reshape(1, CH)

    dn_p, de_p = _hist(nidx_w, hidx_w)
    dn_s, de_i = _scales(dn_p.reshape(NW, N_NODES), de_p.reshape(NW, N_EDGES))
    h = _proj(x, wt, b2, dn_s)
    e_p = _segpass(h, nidx_c, hidx_c)
    e = _combine(e_p, de_i, relu=False)
    y_p = _segpass(e, hidx_c, nidx_c)
    y = _combine(y_p, dn_s, relu=True)
    return y
